# baseline (speedup 1.0000x reference)
import functools

import jax
import jax.numpy as jnp
from jax.experimental import pallas as pl
from jax.experimental.pallas import tpu as pltpu


def _mm_relu_body(h_ref, w_ref, b_ref, o_ref):
    o_ref[...] = jnp.maximum(
        jnp.dot(h_ref[...], w_ref[...], preferred_element_type=jnp.float32)
        + b_ref[...][None, :], 0.0)


def _mm_combine_body(h_ref, agg_ref, ws_ref, wn_ref, b_ref, o_ref, *, act):
    out = (jnp.dot(h_ref[...], ws_ref[...], preferred_element_type=jnp.float32)
           + jnp.dot(agg_ref[...], wn_ref[...], preferred_element_type=jnp.float32)
           + b_ref[...][None, :])
    if act:
        out = jnp.tanh(out + out)
    o_ref[...] = out


def _mm_relu(h, W, b):
    n, _ = h.shape
    dout = W.shape[1]
    return pl.pallas_call(
        _mm_relu_body,
        out_shape=jax.ShapeDtypeStruct((n, dout), jnp.float32),
    )(h, W, b)


def _mm_combine(h, agg, Ws, Wn, b, act):
    n, _ = h.shape
    dout = Ws.shape[1]
    return pl.pallas_call(
        functools.partial(_mm_combine_body, act=act),
        out_shape=jax.ShapeDtypeStruct((n, dout), jnp.float32),
    )(h, agg, Ws, Wn, b)


def _segmax_body(edge_ref, m_ref, o_ref):
    ch = edge_ref.shape[1]

    @pl.when(pl.program_id(0) == 0)
    def _init():
        o_ref[...] = jnp.zeros_like(o_ref)

    def body(i, _):
        s = edge_ref[0, i]
        d = edge_ref[1, i]
        row = m_ref[pl.ds(s, 1), :]
        cur = o_ref[pl.ds(d, 1), :]
        o_ref[pl.ds(d, 1), :] = jnp.maximum(cur, row)
        return 0

    jax.lax.fori_loop(0, ch, body, 0)


def _segmax(m, edge_index):
    n, f = m.shape
    e = edge_index.shape[1]
    ch = 1280 if e % 1280 == 0 else e
    grid = e // ch
    return pl.pallas_call(
        _segmax_body,
        grid=(grid,),
        in_specs=[
            pl.BlockSpec((2, ch), lambda i: (0, i), memory_space=pltpu.SMEM),
            pl.BlockSpec((n, f), lambda i: (0, 0)),
        ],
        out_specs=pl.BlockSpec((n, f), lambda i: (0, 0)),
        out_shape=jax.ShapeDtypeStruct((n, f), jnp.float32),
    )(edge_index, m)


def kernel(inputs, edge_index, Wp1, bp1, Ws1, Wn1, b1, Wp2, bp2, Ws2, Wn2,
           b2, Wp3, bp3, Ws3, Wn3, b3):
    def sage(h, Wp, bp, Ws, Wn, b, act):
        m = _mm_relu(h, Wp, bp)
        agg = _segmax(m, edge_index)
        return _mm_combine(h, agg, Ws, Wn, b, act)

    h = sage(inputs, Wp1, bp1, Ws1, Wn1, b1, True)
    h = sage(h, Wp2, bp2, Ws2, Wn2, b2, True)
    h = sage(h, Wp2, bp2, Ws2, Wn2, b2, True)
    h = sage(h, Wp3, bp3, Ws3, Wn3, b3, False)
    return h
